# baseline (speedup 1.0000x reference)
import functools
import jax
import jax.numpy as jnp
import numpy as np
from jax.experimental import pallas as pl
from jax.experimental.pallas import tpu as pltpu

CUTOFF = 5.0
SBF_DIM = 7
RBF_DIM = 16
HEADS = 16


def _envelope(d):
    xs = d / CUTOFF
    return 1.0 - 21.0 * xs**5 + 35.0 * xs**6 - 15.0 * xs**7


def _mm_kernel(x_ref, w_ref, b_ref, o_ref, *, act):
    y = jnp.dot(x_ref[...], w_ref[...], preferred_element_type=jnp.float32)
    y = y + b_ref[...]
    if act == "silu":
        y = y * jax.nn.sigmoid(y)
    elif act == "sigmoid":
        y = jax.nn.sigmoid(y)
    o_ref[...] = y


def _mm(x, w, b=None, act="none", be=512):
    E, K = x.shape
    N = w.shape[1]
    if b is None:
        b = jnp.zeros((N,), jnp.float32)
    pad = (-E) % be
    if pad:
        x = jnp.pad(x, ((0, pad), (0, 0)))
    G = x.shape[0] // be
    out = pl.pallas_call(
        functools.partial(_mm_kernel, act=act),
        grid=(G,),
        in_specs=[
            pl.BlockSpec((be, K), lambda i: (i, 0)),
            pl.BlockSpec((K, N), lambda i: (0, 0)),
            pl.BlockSpec((N,), lambda i: (0,)),
        ],
        out_specs=pl.BlockSpec((be, N), lambda i: (i, 0)),
        out_shape=jax.ShapeDtypeStruct((x.shape[0], N), jnp.float32),
    )(x, w, b)
    return out[:E]


def _build_triplets(src, num_nodes, t_cap):
    num_edges = src.shape[0]
    order = jnp.argsort(src)
    counts = jnp.bincount(src, length=num_nodes)
    offsets = jnp.cumsum(counts) - counts
    pc = counts * counts
    cpc = jnp.cumsum(pc)
    start_all = cpc - pc
    t = jnp.arange(t_cap)
    node = jnp.minimum(jnp.searchsorted(cpc, t, side='right'), num_nodes - 1)
    valid = t < cpc[-1]
    local = t - start_all[node]
    c = jnp.maximum(counts[node], 1)
    a = local // c
    b = local % c
    i1 = jnp.clip(offsets[node] + a, 0, num_edges - 1)
    i2 = jnp.clip(offsets[node] + b, 0, num_edges - 1)
    e1 = order[i1]
    e2 = order[i2]
    mask = valid & (e1 != e2)
    return e2, e1, mask


def kernel(atom_pos, edge_attr, atom_emb, W_mat, b_mat, W_emb, b_emb, Wq, Wk, Wv, Wsbf, Wea, Wo, Wg, W1, W2, edge_index, x):
    N = atom_pos.shape[0]
    src = edge_index[0]
    dst = edge_index[1]
    T_CAP = 16 * edge_index.shape[1]
    neo_src, neo_dst, tmask = _build_triplets(src, N, T_CAP)
    diff = atom_pos[src] - atom_pos[dst]
    bond_d = jnp.sqrt(jnp.sum(diff * diff, axis=1) + 1e-12)
    env = _envelope(bond_d)[:, None]
    atom_j = src[neo_dst]
    atom_i = dst[neo_dst]
    atom_k = dst[neo_src]
    neo_x = edge_attr * env
    neo_x = _mm(neo_x, W_mat, b_mat, act="silu")
    atom_embeddings = atom_emb[x]
    neo_edge_attr = atom_embeddings[atom_j]
    ji = atom_pos[atom_i] - atom_pos[atom_j]
    jk = atom_pos[atom_k] - atom_pos[atom_j]
    cosang = jnp.sum(ji * jk, axis=1)
    cr = jnp.cross(ji, jk)
    sinang = jnp.sqrt(jnp.sum(cr * cr, axis=1) + 1e-12)
    angle = jnp.arctan2(sinang, cosang)
    d_t = bond_d[neo_src]
    n_r = jnp.arange(1, RBF_DIM + 1, dtype=jnp.float32)
    rad = jnp.sqrt(2.0 / CUTOFF) * jnp.sin(n_r[None, :] * jnp.pi * d_t[:, None] / CUTOFF) / d_t[:, None]
    rad = rad * _envelope(d_t)[:, None]
    m_a = jnp.arange(SBF_DIM, dtype=jnp.float32)
    ang = jnp.cos(m_a[None, :] * angle[:, None])
    edge_sbf = (ang[:, :, None] * rad[:, None, :]).reshape(-1, SBF_DIM * RBF_DIM)
    node_rbf = jnp.sqrt(2.0 / CUTOFF) * jnp.sin(n_r[None, :] * jnp.pi * bond_d[:, None] / CUTOFF) / bond_d[:, None]
    node_rbf = node_rbf * env
    h = _mm(neo_x, W_emb, b_emb, act="silu")
    E = h.shape[0]
    dh = h.shape[1] // HEADS
    scale = 1.0 / jnp.sqrt(float(dh))
    for l in range(Wq.shape[0]):
        qkv = _mm(h, jnp.concatenate([Wq[l], Wk[l], Wv[l]], axis=1))
        q, k, v = jnp.split(qkv, 3, axis=1)
        q = q.reshape(E, HEADS, dh)
        k = k.reshape(E, HEADS, dh)
        v = v.reshape(E, HEADS, dh)
        ea = _mm(neo_edge_attr, Wea[l]).reshape(-1, HEADS, dh)
        ks = k[neo_src] + ea
        vs = v[neo_src] + ea
        logits = jnp.sum(q[neo_dst] * ks, axis=-1) * scale + edge_sbf @ Wsbf[l]
        logits = jnp.where(tmask[:, None], logits, -jnp.inf)
        mx = jax.ops.segment_max(logits, neo_dst, num_segments=E)
        w = jnp.where(tmask[:, None], jnp.exp(logits - mx[neo_dst]), 0.0)
        den = jax.ops.segment_sum(w, neo_dst, num_segments=E)
        attn = w / (den[neo_dst] + 1e-16)
        agg = jax.ops.segment_sum(attn[:, :, None] * vs, neo_dst, num_segments=E).reshape(E, HEADS * dh)
        gate = jax.nn.sigmoid(node_rbf @ Wg[l])
        h = h + _mm(agg, Wo[l], act="silu") * gate
    atom_feat = jax.ops.segment_sum(h, src, num_segments=N)
    graph_feat = jnp.mean(atom_feat, axis=0, keepdims=True)
    out = jax.nn.silu(graph_feat @ W1) @ W2
    return out


# speedup vs baseline: 87.4592x; 87.4592x over previous
import functools
import jax
import jax.numpy as jnp
import numpy as np
from jax import lax
from jax.experimental import pallas as pl
from jax.experimental.pallas import tpu as pltpu
from jax.experimental.pallas import tpu_sc as plsc

CUTOFF = 5.0
SBF_DIM = 7
RBF_DIM = 16
HEADS = 16
DH = 16

NWORK = 32
GO = 32
SRCW = 640
DSTW = 544


def _envelope(d):
    xs = d / CUTOFF
    return 1.0 - 21.0 * xs**5 + 35.0 * xs**6 - 15.0 * xs**7


def _mm_kernel(x_ref, w_ref, b_ref, o_ref, *, act):
    y = jnp.dot(x_ref[...], w_ref[...], preferred_element_type=jnp.float32)
    y = y + b_ref[...]
    if act == "silu":
        y = y * jax.nn.sigmoid(y)
    elif act == "sigmoid":
        y = jax.nn.sigmoid(y)
    o_ref[...] = y


def _mm(x, w, b=None, act="none", be=512):
    E, K = x.shape
    N = w.shape[1]
    if b is None:
        b = jnp.zeros((N,), jnp.float32)
    pad = (-E) % be
    if pad:
        x = jnp.pad(x, ((0, pad), (0, 0)))
    G = x.shape[0] // be
    out = pl.pallas_call(
        functools.partial(_mm_kernel, act=act),
        grid=(G,),
        in_specs=[
            pl.BlockSpec((be, K), lambda i: (i, 0)),
            pl.BlockSpec((K, N), lambda i: (0, 0)),
            pl.BlockSpec((N,), lambda i: (0,)),
        ],
        out_specs=pl.BlockSpec((be, N), lambda i: (i, 0)),
        out_shape=jax.ShapeDtypeStruct((x.shape[0], N), jnp.float32),
    )(x, w, b)
    return out[:E]


def _make_sc_attn(EP):
    NP = EP // NWORK
    NG = NP // GO
    mesh = plsc.VectorSubcoreMesh(core_axis_name="c", subcore_axis_name="s")

    @functools.partial(
        pl.kernel,
        mesh=mesh,
        out_type=jax.ShapeDtypeStruct((EP * HEADS * DH,), jnp.float32),
        scratch_types=[
            pltpu.VMEM((24 * SRCW,), jnp.float32),
            pltpu.VMEM((GO * DSTW,), jnp.float32),
            pltpu.VMEM((16,), jnp.float32),
            pltpu.VMEM((256,), jnp.float32),
            pltpu.VMEM((GO * HEADS * DH,), jnp.float32),
        ],
    )
    def sc_attn(src_hbm, dst_hbm, out_hbm,
                srcbuf, dstbuf, den, acc, outbuf):
        zeros16 = jnp.zeros((16,), jnp.float32)
        wid = lax.axis_index("s") * 2 + lax.axis_index("c")
        p_base = wid * NP

        def group_body(g, _):
            p0 = p_base + g * GO
            pltpu.sync_copy(dst_hbm.at[pl.ds(p0 * DSTW, GO * DSTW)], dstbuf)

            def run_body(lp, _):
                dbase = lp * DSTW
                qea = dstbuf[pl.ds(dbase + 512, 16)]
                dvec = dstbuf[pl.ds(dbase + 528, 16)]
                dxe = dvec[0]
                dye = dvec[1]
                dze = dvec[2]
                d_e = dvec[3]
                off = lax.convert_element_type(dvec[4], jnp.int32)
                ln = lax.convert_element_type(dvec[5], jnp.int32)
                a_self = lax.convert_element_type(dvec[6], jnp.int32)
                den[...] = zeros16
                for h in range(HEADS):
                    acc[pl.ds(h * 16, 16)] = zeros16

                nch = (ln + 15) // 16

                def chunk_body(j, _):
                    base = off + j * 16
                    base8 = jnp.minimum((base // 8) * 8, EP - 24)
                    shift = base - base8
                    pltpu.sync_copy(
                        src_hbm.at[pl.ds(base8 * SRCW, 24 * SRCW)], srcbuf)
                    cnt = jnp.minimum(ln - j * 16, 16)

                    def trip_body(b, _b):
                        rbase = (shift + b) * SRCW
                        dot = zeros16
                        for d in range(DH):
                            kd = srcbuf[pl.ds(rbase + d * 16, 16)]
                            dot = dot + dstbuf[pl.ds(dbase + d * 16, 16)] * kd
                        logit = dot + qea
                        sdv = srcbuf[pl.ds(rbase + 624, 16)]
                        dxs = sdv[0]
                        dys = sdv[1]
                        dzs = sdv[2]
                        d_s = sdv[3]
                        cosang = dxe * dxs + dye * dys + dze * dzs
                        uv = jnp.full((16,), cosang) / jnp.full((16,), d_e * d_s)
                        u = jnp.minimum(jnp.maximum(uv, -1.0), 1.0)
                        tm1 = jnp.ones((16,), jnp.float32)
                        t = u
                        logit = logit + srcbuf[pl.ds(rbase + 512, 16)]
                        logit = logit + t * srcbuf[pl.ds(rbase + 512 + 16, 16)]
                        for m in range(2, SBF_DIM):
                            tm1, t = t, 2.0 * u * t - tm1
                            logit = logit + t * srcbuf[pl.ds(rbase + 512 + m * 16, 16)]
                        w = jnp.exp(logit)

                        @pl.when(j * 16 + b != a_self)
                        def _upd():
                            den[...] = den[...] + w
                            for h in range(HEADS):
                                wh = jnp.full((16,), w[h])
                                acc[pl.ds(h * 16, 16)] = acc[pl.ds(h * 16, 16)] + wh * srcbuf[pl.ds(rbase + 256 + h * 16, 16)]
                        return None

                    lax.fori_loop(0, cnt, trip_body, None)
                    return None

                lax.fori_loop(0, nch, chunk_body, None)

                dv = den[...]
                obase = lp * (HEADS * DH)
                for h in range(HEADS):
                    dh_s = dv[h]
                    ea_h = dstbuf[pl.ds(dbase + 256 + h * 16, 16)]
                    outbuf[pl.ds(obase + h * 16, 16)] = (
                        acc[pl.ds(h * 16, 16)] + jnp.full((16,), dh_s) * ea_h
                    ) / jnp.full((16,), dh_s + 1e-16)
                return None

            lax.fori_loop(0, GO, run_body, None)
            pltpu.sync_copy(
                outbuf, out_hbm.at[pl.ds(p0 * (HEADS * DH), GO * HEADS * DH)])
            return None

        lax.fori_loop(0, NG, group_body, None)

    return sc_attn


def kernel(atom_pos, edge_attr, atom_emb, W_mat, b_mat, W_emb, b_emb, Wq, Wk, Wv, Wsbf, Wea, Wo, Wg, W1, W2, edge_index, x):
    N = atom_pos.shape[0]
    E = edge_index.shape[1]
    src = edge_index[0]
    dst = edge_index[1]
    T_CAP = 16 * E

    order = jnp.argsort(src)
    src_s = src[order]
    dst_s = dst[order]
    counts = jnp.bincount(src, length=N)
    offsets = jnp.cumsum(counts) - counts
    pc = counts * counts
    cpc = jnp.cumsum(pc)
    start_all = cpc - pc
    off_p = offsets[src_s].astype(jnp.int32)
    a_p = (jnp.arange(E, dtype=jnp.int32) - off_p).astype(jnp.int32)
    c_p = counts[src_s].astype(jnp.int32)
    t0_p = start_all[src_s].astype(jnp.int32) + a_p * c_p
    lim = jnp.minimum(cpc[-1], T_CAP).astype(jnp.int32)
    len_p = jnp.clip(jnp.minimum(t0_p + c_p, lim) - t0_p, 0, None).astype(jnp.int32)

    diff_s = atom_pos[src_s] - atom_pos[dst_s]
    bond_d = jnp.sqrt(jnp.sum(diff_s * diff_s, axis=1) + 1e-12)
    env = _envelope(bond_d)[:, None]
    n_r = jnp.arange(1, RBF_DIM + 1, dtype=jnp.float32)
    node_rbf = jnp.sqrt(2.0 / CUTOFF) * jnp.sin(n_r[None, :] * jnp.pi * bond_d[:, None] / CUTOFF) / bond_d[:, None]
    node_rbf = node_rbf * env

    emb_s = atom_emb[x[src_s]]

    neo_x = _mm(edge_attr[order] * env, W_mat, b_mat, act="silu")
    h = _mm(neo_x, W_emb, b_emb, act="silu")

    scale = 1.0 / jnp.sqrt(float(DH))
    EP = ((E + NWORK * GO * 8 - 1) // (NWORK * GO * 8)) * NWORK * GO * 8
    rpad = EP - E
    sc_attn = _make_sc_attn(EP)

    meta_f = jnp.stack(
        [off_p.astype(jnp.float32), len_p.astype(jnp.float32), a_p.astype(jnp.float32)],
        axis=1)
    zcol = jnp.zeros((E, 12), jnp.float32)
    zcol9 = jnp.zeros((E, 9), jnp.float32)

    L = Wq.shape[0]
    for l in range(L):
        qkv = _mm(h, jnp.concatenate([Wq[l], Wk[l], Wv[l]], axis=1))
        q, k, v = jnp.split(qkv, 3, axis=1)
        ea = _mm(emb_s, Wea[l])
        qs = q * scale
        qea = jnp.sum((qs.reshape(E, HEADS, DH) * ea.reshape(E, HEADS, DH)), axis=2)
        Wsbf_t = jnp.transpose(Wsbf[l].reshape(SBF_DIM, RBF_DIM, HEADS), (1, 0, 2)).reshape(RBF_DIM, SBF_DIM * HEADS)
        G = _mm(node_rbf, Wsbf_t)
        qs_T = qs.reshape(E, HEADS, DH).transpose(0, 2, 1).reshape(E, HEADS * DH)
        k_T = k.reshape(E, HEADS, DH).transpose(0, 2, 1).reshape(E, HEADS * DH)
        srctab = jnp.concatenate([k_T, v, G, diff_s, bond_d[:, None], zcol], axis=1)
        dsttab = jnp.concatenate([qs_T, ea, qea, diff_s, bond_d[:, None], meta_f, zcol9], axis=1)
        srctab = jnp.pad(srctab, ((0, rpad), (0, 0))).reshape(-1)
        dsttab = jnp.pad(dsttab, ((0, rpad), (0, 0))).reshape(-1)
        agg = sc_attn(srctab, dsttab)
        agg = agg.reshape(EP, HEADS * DH)[:E]
        gate = _mm(node_rbf, Wg[l], act="sigmoid")
        h = h + _mm(agg, Wo[l], act="silu") * gate

    graph_feat = (jnp.sum(h, axis=0, keepdims=True) / N)
    out = jax.nn.silu(graph_feat @ W1) @ W2
    return out


# speedup vs baseline: 134.1998x; 1.5344x over previous
import functools
import jax
import jax.numpy as jnp
import numpy as np
from jax import lax
from jax.experimental import pallas as pl
from jax.experimental.pallas import tpu as pltpu
from jax.experimental.pallas import tpu_sc as plsc

CUTOFF = 5.0
SBF_DIM = 7
RBF_DIM = 16
HEADS = 16
DH = 16

NWORK = 32
GO = 32
SRCW = 640
DSTW = 544


def _envelope(d):
    xs = d / CUTOFF
    return 1.0 - 21.0 * xs**5 + 35.0 * xs**6 - 15.0 * xs**7


def _mm_kernel(x_ref, w_ref, b_ref, o_ref, *, act):
    y = jnp.dot(x_ref[...], w_ref[...], preferred_element_type=jnp.float32)
    y = y + b_ref[...]
    if act == "silu":
        y = y * jax.nn.sigmoid(y)
    elif act == "sigmoid":
        y = jax.nn.sigmoid(y)
    o_ref[...] = y


def _mm(x, w, b=None, act="none", be=512):
    E, K = x.shape
    N = w.shape[1]
    if b is None:
        b = jnp.zeros((N,), jnp.float32)
    pad = (-E) % be
    if pad:
        x = jnp.pad(x, ((0, pad), (0, 0)))
    G = x.shape[0] // be
    out = pl.pallas_call(
        functools.partial(_mm_kernel, act=act),
        grid=(G,),
        in_specs=[
            pl.BlockSpec((be, K), lambda i: (i, 0)),
            pl.BlockSpec((K, N), lambda i: (0, 0)),
            pl.BlockSpec((N,), lambda i: (0,)),
        ],
        out_specs=pl.BlockSpec((be, N), lambda i: (i, 0)),
        out_shape=jax.ShapeDtypeStruct((x.shape[0], N), jnp.float32),
    )(x, w, b)
    return out[:E]


def _make_sc_attn(EP):
    NP = EP // NWORK
    NG = NP // GO
    mesh = plsc.VectorSubcoreMesh(core_axis_name="c", subcore_axis_name="s")

    @functools.partial(
        pl.kernel,
        mesh=mesh,
        out_type=jax.ShapeDtypeStruct((EP * HEADS * DH,), jnp.float32),
        scratch_types=[
            pltpu.VMEM((24 * SRCW,), jnp.float32),
            pltpu.VMEM((GO * DSTW,), jnp.float32),
            pltpu.VMEM((16,), jnp.float32),
            pltpu.VMEM((256,), jnp.float32),
            pltpu.VMEM((GO * HEADS * DH,), jnp.float32),
            pltpu.SMEM((1,), jnp.int32),
        ],
    )
    def sc_attn(src_hbm, dst_hbm, out_hbm,
                srcbuf, dstbuf, den, acc, outbuf, lastb):
        zeros16 = jnp.zeros((16,), jnp.float32)
        wid = lax.axis_index("s") * 2 + lax.axis_index("c")
        p_base = wid * NP
        lastb[0] = -1

        def group_body(g, _):
            p0 = p_base + g * GO
            pltpu.sync_copy(dst_hbm.at[pl.ds(p0 * DSTW, GO * DSTW)], dstbuf)

            def run_body(lp, _):
                dbase = lp * DSTW
                qea = dstbuf[pl.ds(dbase + 512, 16)]
                dvec = dstbuf[pl.ds(dbase + 528, 16)]
                dxe = dvec[0]
                dye = dvec[1]
                dze = dvec[2]
                d_e = dvec[3]
                off = lax.convert_element_type(dvec[4], jnp.int32)
                ln = lax.convert_element_type(dvec[5], jnp.int32)
                a_self = lax.convert_element_type(dvec[6], jnp.int32)
                den[...] = zeros16
                for h in range(HEADS):
                    acc[pl.ds(h * 16, 16)] = zeros16

                nch = (ln + 15) // 16

                def chunk_body(j, _):
                    base = off + j * 16
                    base8 = jnp.minimum((base // 8) * 8, EP - 24)
                    shift = base - base8

                    @pl.when(base8 != lastb[0])
                    def _stage():
                        pltpu.sync_copy(
                            src_hbm.at[pl.ds(base8 * SRCW, 24 * SRCW)], srcbuf)
                        lastb[0] = base8

                    cnt = jnp.minimum(ln - j * 16, 16)

                    def trip_body(b, _b):
                        rbase = (shift + b) * SRCW
                        dot = zeros16
                        for d in range(DH):
                            kd = srcbuf[pl.ds(rbase + d * 16, 16)]
                            dot = dot + dstbuf[pl.ds(dbase + d * 16, 16)] * kd
                        logit = dot + qea
                        sdv = srcbuf[pl.ds(rbase + 624, 16)]
                        dxs = sdv[0]
                        dys = sdv[1]
                        dzs = sdv[2]
                        d_s = sdv[3]
                        cosang = dxe * dxs + dye * dys + dze * dzs
                        uv = jnp.full((16,), cosang) / jnp.full((16,), d_e * d_s)
                        u = jnp.minimum(jnp.maximum(uv, -1.0), 1.0)
                        tm1 = jnp.ones((16,), jnp.float32)
                        t = u
                        logit = logit + srcbuf[pl.ds(rbase + 512, 16)]
                        logit = logit + t * srcbuf[pl.ds(rbase + 512 + 16, 16)]
                        for m in range(2, SBF_DIM):
                            tm1, t = t, 2.0 * u * t - tm1
                            logit = logit + t * srcbuf[pl.ds(rbase + 512 + m * 16, 16)]
                        w = jnp.exp(logit)

                        @pl.when(j * 16 + b != a_self)
                        def _upd():
                            den[...] = den[...] + w
                            for h in range(HEADS):
                                wh = jnp.full((16,), w[h])
                                acc[pl.ds(h * 16, 16)] = acc[pl.ds(h * 16, 16)] + wh * srcbuf[pl.ds(rbase + 256 + h * 16, 16)]
                        return None

                    lax.fori_loop(0, cnt, trip_body, None)
                    return None

                lax.fori_loop(0, nch, chunk_body, None)

                dv = den[...]
                obase = lp * (HEADS * DH)
                for h in range(HEADS):
                    dh_s = dv[h]
                    ea_h = dstbuf[pl.ds(dbase + 256 + h * 16, 16)]
                    outbuf[pl.ds(obase + h * 16, 16)] = (
                        acc[pl.ds(h * 16, 16)] + jnp.full((16,), dh_s) * ea_h
                    ) / jnp.full((16,), dh_s + 1e-16)
                return None

            lax.fori_loop(0, GO, run_body, None)
            pltpu.sync_copy(
                outbuf, out_hbm.at[pl.ds(p0 * (HEADS * DH), GO * HEADS * DH)])
            return None

        lax.fori_loop(0, NG, group_body, None)

    return sc_attn


def kernel(atom_pos, edge_attr, atom_emb, W_mat, b_mat, W_emb, b_emb, Wq, Wk, Wv, Wsbf, Wea, Wo, Wg, W1, W2, edge_index, x):
    N = atom_pos.shape[0]
    E = edge_index.shape[1]
    src = edge_index[0]
    dst = edge_index[1]
    T_CAP = 16 * E

    order = jnp.argsort(src)
    src_s = src[order]
    dst_s = dst[order]
    counts = jnp.bincount(src, length=N)
    offsets = jnp.cumsum(counts) - counts
    pc = counts * counts
    cpc = jnp.cumsum(pc)
    start_all = cpc - pc
    off_p = offsets[src_s].astype(jnp.int32)
    a_p = (jnp.arange(E, dtype=jnp.int32) - off_p).astype(jnp.int32)
    c_p = counts[src_s].astype(jnp.int32)
    t0_p = start_all[src_s].astype(jnp.int32) + a_p * c_p
    lim = jnp.minimum(cpc[-1], T_CAP).astype(jnp.int32)
    len_p = jnp.clip(jnp.minimum(t0_p + c_p, lim) - t0_p, 0, None).astype(jnp.int32)

    diff_s = atom_pos[src_s] - atom_pos[dst_s]
    bond_d = jnp.sqrt(jnp.sum(diff_s * diff_s, axis=1) + 1e-12)
    env = _envelope(bond_d)[:, None]
    n_r = jnp.arange(1, RBF_DIM + 1, dtype=jnp.float32)
    node_rbf = jnp.sqrt(2.0 / CUTOFF) * jnp.sin(n_r[None, :] * jnp.pi * bond_d[:, None] / CUTOFF) / bond_d[:, None]
    node_rbf = node_rbf * env

    emb_s = atom_emb[x[src_s]]

    neo_x = _mm(edge_attr[order] * env, W_mat, b_mat, act="silu")
    h = _mm(neo_x, W_emb, b_emb, act="silu")

    scale = 1.0 / jnp.sqrt(float(DH))
    EP = ((E + NWORK * GO * 8 - 1) // (NWORK * GO * 8)) * NWORK * GO * 8
    rpad = EP - E
    sc_attn = _make_sc_attn(EP)

    meta_f = jnp.stack(
        [off_p.astype(jnp.float32), len_p.astype(jnp.float32), a_p.astype(jnp.float32)],
        axis=1)
    zcol = jnp.zeros((E, 12), jnp.float32)
    zcol9 = jnp.zeros((E, 9), jnp.float32)

    def _perm_heads(W):
        return W.reshape(-1, HEADS, DH).transpose(0, 2, 1).reshape(-1, HEADS * DH)

    L = Wq.shape[0]
    for l in range(L):
        qkv = _mm(h, jnp.concatenate(
            [_perm_heads(Wq[l]) * scale, _perm_heads(Wk[l]), Wv[l]], axis=1))
        qs_T, k_T, v = jnp.split(qkv, 3, axis=1)
        eas = _mm(emb_s, jnp.concatenate([Wea[l], _perm_heads(Wea[l])], axis=1))
        ea, ea_T = jnp.split(eas, 2, axis=1)
        qea = jnp.sum((qs_T.reshape(E, DH, HEADS) * ea_T.reshape(E, DH, HEADS)), axis=1)
        Wsbf_t = jnp.transpose(Wsbf[l].reshape(SBF_DIM, RBF_DIM, HEADS), (1, 0, 2)).reshape(RBF_DIM, SBF_DIM * HEADS)
        G = _mm(node_rbf, Wsbf_t)
        srctab = jnp.concatenate([k_T, v, G, diff_s, bond_d[:, None], zcol], axis=1)
        dsttab = jnp.concatenate([qs_T, ea, qea, diff_s, bond_d[:, None], meta_f, zcol9], axis=1)
        srctab = jnp.pad(srctab, ((0, rpad), (0, 0))).reshape(-1)
        dsttab = jnp.pad(dsttab, ((0, rpad), (0, 0))).reshape(-1)
        agg = sc_attn(srctab, dsttab)
        agg = agg.reshape(EP, HEADS * DH)[:E]
        gate = _mm(node_rbf, Wg[l], act="sigmoid")
        h = h + _mm(agg, Wo[l], act="silu") * gate

    graph_feat = (jnp.sum(h, axis=0, keepdims=True) / N)
    out = jax.nn.silu(graph_feat @ W1) @ W2
    return out


# speedup vs baseline: 173.4858x; 1.2927x over previous
import functools
import jax
import jax.numpy as jnp
import numpy as np
from jax import lax
from jax.experimental import pallas as pl
from jax.experimental.pallas import tpu as pltpu
from jax.experimental.pallas import tpu_sc as plsc

CUTOFF = 5.0
SBF_DIM = 7
RBF_DIM = 16
HEADS = 16
DH = 16

NWORK = 32
GO = 32
SRCW = 640
DSTW = 544


def _envelope(d):
    xs = d / CUTOFF
    return 1.0 - 21.0 * xs**5 + 35.0 * xs**6 - 15.0 * xs**7


def _mm_kernel(x_ref, w_ref, b_ref, o_ref, *, act):
    y = jnp.dot(x_ref[...], w_ref[...], preferred_element_type=jnp.float32)
    y = y + b_ref[...]
    if act == "silu":
        y = y * jax.nn.sigmoid(y)
    elif act == "sigmoid":
        y = jax.nn.sigmoid(y)
    o_ref[...] = y


def _tab_kernel(h_ref, emb_ref, rbf_ref, geo_ref, wqkv_ref, weas_ref, wsbf_ref,
                src_ref, dst_ref):
    qkv = jnp.dot(h_ref[...], wqkv_ref[...], preferred_element_type=jnp.float32)
    eas = jnp.dot(emb_ref[...], weas_ref[...], preferred_element_type=jnp.float32)
    G = jnp.dot(rbf_ref[...], wsbf_ref[...], preferred_element_type=jnp.float32)
    geo = geo_ref[...]
    src_ref[:, 0:512] = qkv[:, 256:768]
    src_ref[:, 512:624] = G
    src_ref[:, 624:640] = geo
    qT = qkv[:, 0:256]
    eaT = eas[:, 256:512]
    dst_ref[:, 0:256] = qT
    dst_ref[:, 256:512] = eas[:, 0:256]
    qea = jnp.zeros((qT.shape[0], HEADS), jnp.float32)
    for d in range(DH):
        qea = qea + qT[:, d * 16:(d + 1) * 16] * eaT[:, d * 16:(d + 1) * 16]
    dst_ref[:, 512:528] = qea
    dst_ref[:, 528:544] = geo


def _tables(h, emb, rbf, geo, wqkv, weas, wsbf, be=512):
    EP = h.shape[0]
    Gr = EP // be
    return pl.pallas_call(
        _tab_kernel,
        grid=(Gr,),
        in_specs=[
            pl.BlockSpec((be, 256), lambda i: (i, 0)),
            pl.BlockSpec((be, 128), lambda i: (i, 0)),
            pl.BlockSpec((be, 16), lambda i: (i, 0)),
            pl.BlockSpec((be, 16), lambda i: (i, 0)),
            pl.BlockSpec((256, 768), lambda i: (0, 0)),
            pl.BlockSpec((128, 512), lambda i: (0, 0)),
            pl.BlockSpec((16, 112), lambda i: (0, 0)),
        ],
        out_specs=[
            pl.BlockSpec((be, SRCW), lambda i: (i, 0)),
            pl.BlockSpec((be, DSTW), lambda i: (i, 0)),
        ],
        out_shape=[
            jax.ShapeDtypeStruct((EP, SRCW), jnp.float32),
            jax.ShapeDtypeStruct((EP, DSTW), jnp.float32),
        ],
    )(h, emb, rbf, geo, wqkv, weas, wsbf)


def _mm(x, w, b=None, act="none", be=512):
    E, K = x.shape
    N = w.shape[1]
    if b is None:
        b = jnp.zeros((N,), jnp.float32)
    pad = (-E) % be
    if pad:
        x = jnp.pad(x, ((0, pad), (0, 0)))
    G = x.shape[0] // be
    out = pl.pallas_call(
        functools.partial(_mm_kernel, act=act),
        grid=(G,),
        in_specs=[
            pl.BlockSpec((be, K), lambda i: (i, 0)),
            pl.BlockSpec((K, N), lambda i: (0, 0)),
            pl.BlockSpec((N,), lambda i: (0,)),
        ],
        out_specs=pl.BlockSpec((be, N), lambda i: (i, 0)),
        out_shape=jax.ShapeDtypeStruct((x.shape[0], N), jnp.float32),
    )(x, w, b)
    return out[:E]


def _make_sc_attn(EP):
    NP = EP // NWORK
    NG = NP // GO
    mesh = plsc.VectorSubcoreMesh(core_axis_name="c", subcore_axis_name="s")

    @functools.partial(
        pl.kernel,
        mesh=mesh,
        out_type=jax.ShapeDtypeStruct((EP * HEADS * DH,), jnp.float32),
        scratch_types=[
            pltpu.VMEM((24 * SRCW,), jnp.float32),
            pltpu.VMEM((GO * DSTW,), jnp.float32),
            pltpu.VMEM((16,), jnp.float32),
            pltpu.VMEM((256,), jnp.float32),
            pltpu.VMEM((GO * HEADS * DH,), jnp.float32),
            pltpu.SMEM((1,), jnp.int32),
        ],
    )
    def sc_attn(src_hbm, dst_hbm, out_hbm,
                srcbuf, dstbuf, den, acc, outbuf, lastb):
        zeros16 = jnp.zeros((16,), jnp.float32)
        wid = lax.axis_index("s") * 2 + lax.axis_index("c")
        p_base = wid * NP
        lastb[0] = -1

        def group_body(g, _):
            p0 = p_base + g * GO
            pltpu.sync_copy(dst_hbm.at[pl.ds(p0 * DSTW, GO * DSTW)], dstbuf)

            def run_body(lp, _):
                dbase = lp * DSTW
                qea = dstbuf[pl.ds(dbase + 512, 16)]
                dvec = dstbuf[pl.ds(dbase + 528, 16)]
                dxe = dvec[0]
                dye = dvec[1]
                dze = dvec[2]
                d_e = dvec[3]
                off = lax.convert_element_type(dvec[4], jnp.int32)
                ln = lax.convert_element_type(dvec[5], jnp.int32)
                a_self = lax.convert_element_type(dvec[6], jnp.int32)
                den[...] = zeros16
                for h in range(HEADS):
                    acc[pl.ds(h * 16, 16)] = zeros16

                nch = (ln + 15) // 16

                def chunk_body(j, _):
                    base = off + j * 16
                    base8 = jnp.minimum((base // 8) * 8, EP - 24)
                    shift = base - base8

                    @pl.when(base8 != lastb[0])
                    def _stage():
                        pltpu.sync_copy(
                            src_hbm.at[pl.ds(base8 * SRCW, 24 * SRCW)], srcbuf)
                        lastb[0] = base8

                    cnt = jnp.minimum(ln - j * 16, 16)

                    def trip_body(b, _b):
                        rbase = (shift + b) * SRCW
                        dot = zeros16
                        for d in range(DH):
                            kd = srcbuf[pl.ds(rbase + d * 16, 16)]
                            dot = dot + dstbuf[pl.ds(dbase + d * 16, 16)] * kd
                        logit = dot + qea
                        sdv = srcbuf[pl.ds(rbase + 624, 16)]
                        dxs = sdv[0]
                        dys = sdv[1]
                        dzs = sdv[2]
                        d_s = sdv[3]
                        cosang = dxe * dxs + dye * dys + dze * dzs
                        uv = jnp.full((16,), cosang) / jnp.full((16,), d_e * d_s)
                        u = jnp.minimum(jnp.maximum(uv, -1.0), 1.0)
                        tm1 = jnp.ones((16,), jnp.float32)
                        t = u
                        logit = logit + srcbuf[pl.ds(rbase + 512, 16)]
                        logit = logit + t * srcbuf[pl.ds(rbase + 512 + 16, 16)]
                        for m in range(2, SBF_DIM):
                            tm1, t = t, 2.0 * u * t - tm1
                            logit = logit + t * srcbuf[pl.ds(rbase + 512 + m * 16, 16)]
                        w = jnp.exp(logit)

                        @pl.when(j * 16 + b != a_self)
                        def _upd():
                            den[...] = den[...] + w
                            for h in range(HEADS):
                                wh = jnp.full((16,), w[h])
                                acc[pl.ds(h * 16, 16)] = acc[pl.ds(h * 16, 16)] + wh * srcbuf[pl.ds(rbase + 256 + h * 16, 16)]
                        return None

                    lax.fori_loop(0, cnt, trip_body, None)
                    return None

                lax.fori_loop(0, nch, chunk_body, None)

                dv = den[...]
                obase = lp * (HEADS * DH)
                for h in range(HEADS):
                    dh_s = dv[h]
                    ea_h = dstbuf[pl.ds(dbase + 256 + h * 16, 16)]
                    outbuf[pl.ds(obase + h * 16, 16)] = (
                        acc[pl.ds(h * 16, 16)] + jnp.full((16,), dh_s) * ea_h
                    ) / jnp.full((16,), dh_s + 1e-16)
                return None

            lax.fori_loop(0, GO, run_body, None)
            pltpu.sync_copy(
                outbuf, out_hbm.at[pl.ds(p0 * (HEADS * DH), GO * HEADS * DH)])
            return None

        lax.fori_loop(0, NG, group_body, None)

    return sc_attn


def kernel(atom_pos, edge_attr, atom_emb, W_mat, b_mat, W_emb, b_emb, Wq, Wk, Wv, Wsbf, Wea, Wo, Wg, W1, W2, edge_index, x):
    N = atom_pos.shape[0]
    E = edge_index.shape[1]
    src = edge_index[0]
    dst = edge_index[1]
    T_CAP = 16 * E

    order = jnp.argsort(src)
    src_s = src[order]
    dst_s = dst[order]
    counts = jnp.bincount(src, length=N)
    offsets = jnp.cumsum(counts) - counts
    pc = counts * counts
    cpc = jnp.cumsum(pc)
    start_all = cpc - pc
    off_p = offsets[src_s].astype(jnp.int32)
    a_p = (jnp.arange(E, dtype=jnp.int32) - off_p).astype(jnp.int32)
    c_p = counts[src_s].astype(jnp.int32)
    t0_p = start_all[src_s].astype(jnp.int32) + a_p * c_p
    lim = jnp.minimum(cpc[-1], T_CAP).astype(jnp.int32)
    len_p = jnp.clip(jnp.minimum(t0_p + c_p, lim) - t0_p, 0, None).astype(jnp.int32)

    diff_s = atom_pos[src_s] - atom_pos[dst_s]
    bond_d = jnp.sqrt(jnp.sum(diff_s * diff_s, axis=1) + 1e-12)
    env = _envelope(bond_d)[:, None]
    n_r = jnp.arange(1, RBF_DIM + 1, dtype=jnp.float32)
    node_rbf = jnp.sqrt(2.0 / CUTOFF) * jnp.sin(n_r[None, :] * jnp.pi * bond_d[:, None] / CUTOFF) / bond_d[:, None]
    node_rbf = node_rbf * env

    emb_s = atom_emb[x[src_s]]

    scale = 1.0 / jnp.sqrt(float(DH))
    EP = ((E + NWORK * GO * 8 - 1) // (NWORK * GO * 8)) * NWORK * GO * 8
    rpad = EP - E
    sc_attn = _make_sc_attn(EP)

    meta_f = jnp.stack(
        [off_p.astype(jnp.float32), len_p.astype(jnp.float32), a_p.astype(jnp.float32)],
        axis=1)
    geo = jnp.concatenate(
        [diff_s, bond_d[:, None], meta_f, jnp.zeros((E, 9), jnp.float32)], axis=1)

    neo_x = _mm(jnp.pad(edge_attr[order] * env, ((0, rpad), (0, 0))), W_mat, b_mat, act="silu")
    h = _mm(neo_x, W_emb, b_emb, act="silu")
    emb_p = jnp.pad(emb_s, ((0, rpad), (0, 0)))
    rbf_p = jnp.pad(node_rbf, ((0, rpad), (0, 0)))
    geo_p = jnp.pad(geo, ((0, rpad), (0, 0)))

    def _perm_heads(W):
        return W.reshape(-1, HEADS, DH).transpose(0, 2, 1).reshape(-1, HEADS * DH)

    L = Wq.shape[0]
    for l in range(L):
        wqkv = jnp.concatenate(
            [_perm_heads(Wq[l]) * scale, _perm_heads(Wk[l]), Wv[l]], axis=1)
        weas = jnp.concatenate([Wea[l], _perm_heads(Wea[l])], axis=1)
        wsbf_t = jnp.transpose(Wsbf[l].reshape(SBF_DIM, RBF_DIM, HEADS), (1, 0, 2)).reshape(RBF_DIM, SBF_DIM * HEADS)
        srctab, dsttab = _tables(h, emb_p, rbf_p, geo_p, wqkv, weas, wsbf_t)
        agg = sc_attn(srctab.reshape(-1), dsttab.reshape(-1))
        agg = agg.reshape(EP, HEADS * DH)
        gate = _mm(rbf_p, Wg[l], act="sigmoid")
        h = h + _mm(agg, Wo[l], act="silu") * gate

    graph_feat = (jnp.sum(h[:E], axis=0, keepdims=True) / N)
    out = jax.nn.silu(graph_feat @ W1) @ W2
    return out
